# baseline (speedup 1.0000x reference)
import functools

import jax
import jax.numpy as jnp
import numpy as np
from jax.experimental import pallas as pl
from jax.experimental.pallas import tpu as pltpu
from jax.experimental.pallas import tpu_sc as plsc

_D = 64
_SEQ = 200
_SCALE = 8.0

_GATHER_WINDOW = 128
_TC_BATCH = 128


def _pe_table() -> np.ndarray:
    half = _D / 2
    positions = np.arange(_SEQ)[:, np.newaxis]
    depths = np.arange(half)[np.newaxis, :] / half
    angle_rads = positions * (1.0 / 10000**depths)
    return np.concatenate(
        [np.sin(angle_rads), np.cos(angle_rads)], axis=-1
    ).astype(np.float32)


def _sc_gather(table, idx):
    n = idx.shape[1]
    mesh = plsc.VectorSubcoreMesh(core_axis_name="core", subcore_axis_name="subcore")

    @functools.partial(
        pl.kernel,
        out_type=jax.ShapeDtypeStruct((n, _D), table.dtype),
        mesh=mesh,
        compiler_params=pltpu.CompilerParams(use_tc_tiling_on_sc=False),
    )
    def k(table_hbm, i_hbm, o_hbm):
        def body(i_vmem, o_vmem):
            pltpu.sync_copy(table_hbm.at[i_vmem.at[0]], o_vmem)

        pltpu.emit_pipeline(
            body,
            grid=(n // _GATHER_WINDOW,),
            in_specs=[pl.BlockSpec((1, _GATHER_WINDOW), index_map=lambda i: (0, i))],
            out_specs=[pl.BlockSpec((_GATHER_WINDOW, _D), index_map=lambda i: (i, 0))],
            core_axis_name=("core", "subcore"),
            dimension_semantics=(pltpu.PARALLEL,),
        )(i_hbm, o_hbm)

    return k(table, idx)


_L_CHUNK = 8


def _fixup_body(g_ref, pe_ref, o_ref, *, b):
    gb = g_ref[...].reshape(_L_CHUNK, b, _D)
    t = jnp.transpose(gb, (0, 2, 1))
    pe = pe_ref[...][:, :, None]
    o_ref[...] = t * _SCALE + pe


def _tc_fixup(g, pe, b):
    return pl.pallas_call(
        functools.partial(_fixup_body, b=b),
        grid=(_SEQ // _L_CHUNK,),
        in_specs=[
            pl.BlockSpec((_L_CHUNK * b, _D), lambda i: (i, 0)),
            pl.BlockSpec((_L_CHUNK, _D), lambda i: (i, 0)),
        ],
        out_specs=pl.BlockSpec((_L_CHUNK, _D, b), lambda i: (i, 0, 0)),
        out_shape=jax.ShapeDtypeStruct((_SEQ, _D, b), jnp.float32),
    )(g, pe)


def kernel(x, table):
    b, l = x.shape
    idx = x.T.reshape(1, b * l).astype(jnp.int32)
    g = _sc_gather(table, idx)
    pe = jnp.asarray(_pe_table())
    out3 = _tc_fixup(g, pe, b)
    return out3.transpose(2, 0, 1)


# speedup vs baseline: 1.0267x; 1.0267x over previous
import functools

import jax
import jax.numpy as jnp
import numpy as np
from jax.experimental import pallas as pl
from jax.experimental.pallas import tpu as pltpu
from jax.experimental.pallas import tpu_sc as plsc

_D = 64
_SEQ = 200
_SCALE = 8.0

_GATHER_WINDOW = 512
_TC_BATCH = 128


def _pe_table() -> np.ndarray:
    half = _D / 2
    positions = np.arange(_SEQ)[:, np.newaxis]
    depths = np.arange(half)[np.newaxis, :] / half
    angle_rads = positions * (1.0 / 10000**depths)
    return np.concatenate(
        [np.sin(angle_rads), np.cos(angle_rads)], axis=-1
    ).astype(np.float32)


def _sc_gather(table, idx):
    n = idx.shape[1]
    mesh = plsc.VectorSubcoreMesh(core_axis_name="core", subcore_axis_name="subcore")

    @functools.partial(
        pl.kernel,
        out_type=jax.ShapeDtypeStruct((n, _D), table.dtype),
        mesh=mesh,
        compiler_params=pltpu.CompilerParams(use_tc_tiling_on_sc=False),
    )
    def k(table_hbm, i_hbm, o_hbm):
        def body(i_vmem, o_vmem):
            pltpu.sync_copy(table_hbm.at[i_vmem.at[0]], o_vmem)

        pltpu.emit_pipeline(
            body,
            grid=(n // _GATHER_WINDOW,),
            in_specs=[pl.BlockSpec((1, _GATHER_WINDOW), index_map=lambda i: (0, i))],
            out_specs=[pl.BlockSpec((_GATHER_WINDOW, _D), index_map=lambda i: (i, 0))],
            core_axis_name=("core", "subcore"),
            dimension_semantics=(pltpu.PARALLEL,),
        )(i_hbm, o_hbm)

    return k(table, idx)


_L_CHUNK = 8


def _fixup_body(g_ref, pe_ref, o_ref, *, b):
    gb = g_ref[...].reshape(_L_CHUNK, b, _D)
    t = jnp.transpose(gb, (0, 2, 1))
    pe = pe_ref[...][:, :, None]
    o_ref[...] = t * _SCALE + pe


def _tc_fixup(g, pe, b):
    return pl.pallas_call(
        functools.partial(_fixup_body, b=b),
        grid=(_SEQ // _L_CHUNK,),
        in_specs=[
            pl.BlockSpec((_L_CHUNK * b, _D), lambda i: (i, 0)),
            pl.BlockSpec((_L_CHUNK, _D), lambda i: (i, 0)),
        ],
        out_specs=pl.BlockSpec((_L_CHUNK, _D, b), lambda i: (i, 0, 0)),
        out_shape=jax.ShapeDtypeStruct((_SEQ, _D, b), jnp.float32),
    )(g, pe)


def kernel(x, table):
    b, l = x.shape
    idx = x.T.reshape(1, b * l).astype(jnp.int32)
    g = _sc_gather(table, idx)
    pe = jnp.asarray(_pe_table())
    out3 = _tc_fixup(g, pe, b)
    return out3.transpose(2, 0, 1)
